# baseline (speedup 1.0000x reference)
import functools

import jax
import jax.numpy as jnp
from jax import lax
from jax.experimental import pallas as pl
from jax.experimental.pallas import tpu as pltpu
from jax.experimental.pallas import tpu_sc as plsc

_PATCH = 16
_LANES = 16
_NC = 2
_NS = 16
_NW = _NC * _NS
_CHUNK = 128


def _src_rows(C, H, W):
    p = _PATCH
    h, w = H // p, W // p
    perm = jax.random.permutation(jax.random.key(42), h * w)
    sh = (perm // w).reshape(h, w)
    sw = (perm % w).reshape(h, w)
    c_b = jnp.arange(C, dtype=jnp.int32)[:, None, None, None]
    r_b = jnp.arange(p, dtype=jnp.int32)[None, None, :, None]
    src = ((c_b * h + sh[None, :, None, :]) * p + r_b) * w + sw[None, :, None, :]
    return src.reshape(-1).astype(jnp.int32)


_K = 16


def _make_gather(num_rows):
    rows_per_w = num_rows // _NW
    n_chunks = rows_per_w // _CHUNK
    n_groups = n_chunks // _K
    bank_rows = _K * _CHUNK
    mesh = plsc.VectorSubcoreMesh(core_axis_name="c", subcore_axis_name="s")

    @functools.partial(
        pl.kernel,
        mesh=mesh,
        out_type=jax.ShapeDtypeStruct((num_rows, _LANES), jnp.float32),
        scratch_types=[
            pltpu.VMEM((n_chunks, _CHUNK), jnp.int32),
            pltpu.VMEM((2, bank_rows, _LANES), jnp.float32),
            pltpu.SemaphoreType.DMA,
            pltpu.SemaphoreType.DMA,
        ],
        compiler_params=pltpu.CompilerParams(use_tc_tiling_on_sc=False),
    )
    def gather(table_hbm, idx_hbm, out_hbm, idx_v, rows_v, sem_g, sem_s):
        wid = lax.axis_index("s") * _NC + lax.axis_index("c")
        base = wid * rows_per_w
        pltpu.sync_copy(idx_hbm.at[wid], idx_v)

        def fire_gathers(g, bank):
            g = jnp.minimum(g, n_groups - 1)
            for k in range(_K):
                pltpu.async_copy(
                    table_hbm.at[idx_v.at[g * _K + k]],
                    rows_v.at[bank, pl.ds(k * _CHUNK, _CHUNK)], sem_g)

        def drain_gathers(bank):
            for k in range(_K):
                pltpu.make_async_copy(
                    table_hbm.at[idx_v.at[0]],
                    rows_v.at[bank, pl.ds(k * _CHUNK, _CHUNK)], sem_g).wait()

        def store(g, bank):
            return pltpu.async_copy(
                rows_v.at[bank],
                out_hbm.at[pl.ds(base + g * bank_rows, bank_rows)], sem_s)

        def drain_store(bank):
            pltpu.make_async_copy(
                rows_v.at[bank], out_hbm.at[pl.ds(base, bank_rows)], sem_s).wait()

        fire_gathers(0, 0)

        def body(i, carry):
            @pl.when(i > 0)
            def _():
                drain_store(1)
            fire_gathers(2 * i + 1, 1)
            drain_gathers(0)
            store(2 * i, 0)
            drain_store(0)
            fire_gathers(2 * i + 2, 0)
            drain_gathers(1)
            store(2 * i + 1, 1)
            return carry

        lax.fori_loop(0, n_groups // 2, body, 0)
        drain_gathers(0)
        drain_store(1)

    return gather


def _to_phys(l, H, W):
    w = W // _PATCH
    c, rem = l // (H * w), l % (H * w)
    h, sw = rem // w, rem % w
    return (((c * (H // 8) + h // 8) * (w // 8) + sw // 8) * 8 + h % 8) * 8 + sw % 8


def kernel(image):
    C, H, W = image.shape
    num_rows = C * H * (W // _PATCH)
    w = W // _PATCH
    t = image.reshape(C, H // 8, 8, W // 128, 8, _LANES)
    table = jnp.transpose(t, (0, 1, 3, 2, 4, 5)).reshape(num_rows, _LANES)
    src_log = _src_rows(C, H, W)
    g = jnp.arange(num_rows, dtype=jnp.int32)
    c, rem = g // (H * w), g % (H * w)
    hh, r2 = rem // 256, rem % 256
    ct, rr, lb = r2 // 64, (r2 % 64) // 8, r2 % 8
    l_of_g = (c * H + hh * 8 + rr) * w + ct * 8 + lb
    phys_src = _to_phys(src_log[l_of_g], H, W).astype(jnp.int32)
    idx = phys_src.reshape(_NW, num_rows // (_NW * _CHUNK), _CHUNK)
    out = _make_gather(num_rows)(table, idx)
    o = out.reshape(C, H // 8, W // 128, 8, 8, _LANES)
    return jnp.transpose(o, (0, 1, 3, 2, 4, 5)).reshape(C, H, W)


# speedup vs baseline: 7.3511x; 7.3511x over previous
import functools

import jax
import jax.numpy as jnp
from jax import lax
from jax.experimental import pallas as pl
from jax.experimental.pallas import tpu as pltpu
from jax.experimental.pallas import tpu_sc as plsc

_PATCH = 16
_LANES = 16
_NC = 2
_NS = 16
_NW = _NC * _NS
_CHUNK = 128


def _src_rows(C, H, W):
    p = _PATCH
    h, w = H // p, W // p
    perm = jax.random.permutation(jax.random.key(42), h * w)
    sh = (perm // w).reshape(h, w)
    sw = (perm % w).reshape(h, w)
    c_b = jnp.arange(C, dtype=jnp.int32)[:, None, None, None]
    r_b = jnp.arange(p, dtype=jnp.int32)[None, None, :, None]
    src = ((c_b * h + sh[None, :, None, :]) * p + r_b) * w + sw[None, :, None, :]
    return src.reshape(-1).astype(jnp.int32)


_K = 16


def _make_gather(num_rows):
    rows_per_w = num_rows // _NW
    n_chunks = rows_per_w // _CHUNK
    n_groups = n_chunks // _K
    bank_rows = _K * _CHUNK
    mesh = plsc.VectorSubcoreMesh(core_axis_name="c", subcore_axis_name="s")

    @functools.partial(
        pl.kernel,
        mesh=mesh,
        out_type=jax.ShapeDtypeStruct((num_rows, _LANES), jnp.float32),
        scratch_types=[
            pltpu.VMEM((n_chunks, _CHUNK), jnp.int32),
            pltpu.VMEM((2, bank_rows, _LANES), jnp.float32),
            pltpu.SemaphoreType.DMA,
            pltpu.SemaphoreType.DMA,
        ],
        compiler_params=pltpu.CompilerParams(use_tc_tiling_on_sc=False),
    )
    def gather(table_hbm, idx_hbm, out_hbm, idx_v, rows_v, sem_g, sem_s):
        wid = lax.axis_index("s") * _NC + lax.axis_index("c")
        base = wid * rows_per_w
        pltpu.sync_copy(idx_hbm.at[wid], idx_v)

        def fire_gathers(g, bank):
            g = jnp.minimum(g, n_groups - 1)
            for k in range(_K):
                pltpu.async_copy(
                    table_hbm.at[idx_v.at[g * _K + k]],
                    rows_v.at[bank, pl.ds(k * _CHUNK, _CHUNK)], sem_g)

        def drain_gathers(bank):
            for k in range(_K):
                pltpu.make_async_copy(
                    table_hbm.at[idx_v.at[0]],
                    rows_v.at[bank, pl.ds(k * _CHUNK, _CHUNK)], sem_g).wait()

        def store(g, bank):
            return pltpu.async_copy(
                rows_v.at[bank],
                out_hbm.at[pl.ds(base + g * bank_rows, bank_rows)], sem_s)

        def drain_store(bank):
            pltpu.make_async_copy(
                rows_v.at[bank], out_hbm.at[pl.ds(base, bank_rows)], sem_s).wait()

        fire_gathers(0, 0)

        def body(i, carry):
            @pl.when(i > 0)
            def _():
                drain_store(1)
            fire_gathers(2 * i + 1, 1)
            drain_gathers(0)
            store(2 * i, 0)
            drain_store(0)
            fire_gathers(2 * i + 2, 0)
            drain_gathers(1)
            store(2 * i + 1, 1)
            return carry

        lax.fori_loop(0, n_groups // 2, body, 0)
        drain_gathers(0)
        drain_store(1)

    return gather


def kernel(image):
    C, H, W = image.shape
    num_rows = C * H * (W // _PATCH)
    table = image.reshape(num_rows, _LANES)
    idx = _src_rows(C, H, W).reshape(_NW, num_rows // (_NW * _CHUNK), _CHUNK)
    out = _make_gather(num_rows)(table, idx)
    return out.reshape(C, H, W)
